# Initial kernel scaffold; baseline (speedup 1.0000x reference)
import jax
import jax.numpy as jnp
from jax.experimental import pallas as pl


def kernel(X, edge_index, W, b, scale):
    raise NotImplementedError("write your pallas kernel here")


# speedup vs baseline: 11.1634x; 11.1634x over previous
import functools

import jax
import jax.numpy as jnp
from jax import lax
from jax.experimental import pallas as pl
from jax.experimental.pallas import tpu as pltpu
from jax.experimental.pallas import tpu_sc as plsc

N = 10000
E = 320000
D = 128

NC = 2
NS = 16
NW = NC * NS
K = 64
CT = 160
NP = 4
CTP = CT // NP
EPT = CT * K
NPAD = 10112
RPT = NPAD // NS
PADIDX = 10008

_mesh = plsc.VectorSubcoreMesh(core_axis_name="c", subcore_axis_name="s")


def _make_deg_kernel(degw):
    @functools.partial(
        pl.kernel,
        out_type=jax.ShapeDtypeStruct((NC, NPAD, degw), jnp.float32),
        mesh=_mesh,
        scratch_types=[
            pltpu.VMEM((CT, K), jnp.int32),
            pltpu.VMEM((K, degw), jnp.float32),
            pltpu.VMEM_SHARED((NPAD, degw), jnp.float32),
        ],
    )
    def _deg_kernel(dst_hbm, ones_hbm, zeros_hbm, degp_hbm, idx_v, ones_v,
                    counts):
        cid = lax.axis_index("c")
        sid = lax.axis_index("s")
        w = cid * NS + sid
        pltpu.sync_copy(dst_hbm.at[w], idx_v)
        pltpu.sync_copy(ones_hbm, ones_v)
        pltpu.sync_copy(zeros_hbm, counts.at[pl.ds(sid * RPT, RPT)])
        plsc.subcore_barrier()

        def body(j, carry):
            pltpu.sync_copy(ones_v, counts.at[idx_v.at[j]], add=True)
            return carry

        lax.fori_loop(0, CT, body, 0)
        plsc.subcore_barrier()
        pltpu.sync_copy(counts.at[pl.ds(sid * RPT, RPT)],
                        degp_hbm.at[cid, pl.ds(sid * RPT, RPT)])

    return _deg_kernel


DEGW = 128
_deg_kernel = _make_deg_kernel(DEGW)


@functools.partial(
    pl.kernel,
    out_type=jax.ShapeDtypeStruct((NC, NPAD, D), jnp.float32),
    mesh=_mesh,
    scratch_types=[
        pltpu.VMEM((CTP, K), jnp.int32),
        pltpu.VMEM((CTP, K), jnp.int32),
        pltpu.VMEM((K, D), jnp.float32),
        pltpu.VMEM((K, D), jnp.float32),
        pltpu.VMEM_SHARED((NPAD, D), jnp.float32),
        pltpu.SemaphoreType.DMA,
        pltpu.SemaphoreType.DMA,
    ],
)
def _scatter_kernel(y_hbm, src_hbm, dst_hbm, zeros_hbm, sp_hbm,
                    src_v, dst_v, rows0, rows1, accum, sem0, sem1):
    cid = lax.axis_index("c")
    sid = lax.axis_index("s")
    w = cid * NS + sid
    pltpu.sync_copy(zeros_hbm, accum.at[pl.ds(sid * RPT, RPT)])
    plsc.subcore_barrier()

    def phase(p, carry):
        pltpu.sync_copy(src_hbm.at[w, pl.ds(p * CTP, CTP)], src_v)
        pltpu.sync_copy(dst_hbm.at[w, pl.ds(p * CTP, CTP)], dst_v)

        def body(jj, c2):
            j0 = 2 * jj
            j1 = j0 + 1
            cp0 = pltpu.async_copy(y_hbm.at[src_v.at[j0]], rows0, sem0)
            cp1 = pltpu.async_copy(y_hbm.at[src_v.at[j1]], rows1, sem1)
            cp0.wait()
            pltpu.sync_copy(rows0, accum.at[dst_v.at[j0]], add=True)
            cp1.wait()
            pltpu.sync_copy(rows1, accum.at[dst_v.at[j1]], add=True)
            return c2

        lax.fori_loop(0, CTP // 2, body, 0)
        return carry

    lax.fori_loop(0, NP, phase, 0)
    plsc.subcore_barrier()
    pltpu.sync_copy(accum.at[pl.ds(sid * RPT, RPT)],
                    sp_hbm.at[cid, pl.ds(sid * RPT, RPT)])


def _linear_body(x_ref, w_ref, d0_ref, d1_ref, y_ref, dis_ref):
    deg = d0_ref[:, :1] + d1_ref[:, :1] + 1.0
    dis = lax.rsqrt(deg)
    xw = jnp.dot(x_ref[...], w_ref[...], preferred_element_type=jnp.float32)
    y_ref[...] = xw * dis
    dis_ref[...] = jnp.broadcast_to(dis, dis_ref.shape)


def _final_body(x_ref, y_ref, s0_ref, s1_ref, dis_ref, b_ref, scale_ref, o_ref):
    x = x_ref[...]
    s = s0_ref[...] + s1_ref[...] + y_ref[...]
    conv = dis_ref[:, :1] * s + b_ref[...]
    mn = jnp.sqrt(jnp.sum(conv * conv, axis=-1, keepdims=True))
    msgn = conv / jnp.maximum(mn, 1e-12)
    xn = jnp.sqrt(jnp.sum(x * x, axis=-1, keepdims=True))
    normed = msgn * xn * scale_ref[0, 0]
    o_ref[...] = 0.5 * normed * (1.0 + lax.erf(normed * 0.7071067811865476))


_BLK = 1000
_GRID = N // _BLK


def kernel(X, edge_index, W, b, scale):
    src = edge_index[0]
    dst = edge_index[1]
    pad = ((0, 0), (0, EPT - E // NW))
    src_t = jnp.pad(src.reshape(NW, E // NW), pad, constant_values=PADIDX)
    dst_t = jnp.pad(dst.reshape(NW, E // NW), pad, constant_values=PADIDX)
    src_t = src_t.reshape(NW, CT, K)
    dst_t = dst_t.reshape(NW, CT, K)

    onesw = jnp.ones((K, DEGW), jnp.float32)
    zerosw = jnp.zeros((RPT, DEGW), jnp.float32)
    zerosD = jnp.zeros((RPT, D), jnp.float32)

    degp = _deg_kernel(dst_t, onesw, zerosw)

    d0 = degp[0, :N, :]
    d1 = degp[1, :N, :]
    y, dis = pl.pallas_call(
        _linear_body,
        grid=(_GRID,),
        in_specs=[
            pl.BlockSpec((_BLK, D), lambda i: (i, 0)),
            pl.BlockSpec((D, D), lambda i: (0, 0)),
            pl.BlockSpec((_BLK, DEGW), lambda i: (i, 0)),
            pl.BlockSpec((_BLK, DEGW), lambda i: (i, 0)),
        ],
        out_specs=[
            pl.BlockSpec((_BLK, D), lambda i: (i, 0)),
            pl.BlockSpec((_BLK, 16), lambda i: (i, 0)),
        ],
        out_shape=[
            jax.ShapeDtypeStruct((N, D), jnp.float32),
            jax.ShapeDtypeStruct((N, 16), jnp.float32),
        ],
    )(X, W, d0, d1)

    y_ext = jnp.concatenate([y, jnp.zeros((NPAD - N, D), jnp.float32)], axis=0)

    sp = _scatter_kernel(y_ext, src_t, dst_t, zerosD)

    out = pl.pallas_call(
        _final_body,
        grid=(_GRID,),
        in_specs=[
            pl.BlockSpec((_BLK, D), lambda i: (i, 0)),
            pl.BlockSpec((_BLK, D), lambda i: (i, 0)),
            pl.BlockSpec((_BLK, D), lambda i: (i, 0)),
            pl.BlockSpec((_BLK, D), lambda i: (i, 0)),
            pl.BlockSpec((_BLK, 16), lambda i: (i, 0)),
            pl.BlockSpec((1, D), lambda i: (0, 0)),
            pl.BlockSpec(memory_space=pltpu.SMEM),
        ],
        out_specs=pl.BlockSpec((_BLK, D), lambda i: (i, 0)),
        out_shape=jax.ShapeDtypeStruct((N, D), jnp.float32),
    )(X, y, sp[0, :N, :], sp[1, :N, :], dis,
      b.reshape(1, D), scale.reshape(1, 1))
    return out


# speedup vs baseline: 11.7598x; 1.0534x over previous
import functools

import jax
import jax.numpy as jnp
from jax import lax
from jax.experimental import pallas as pl
from jax.experimental.pallas import tpu as pltpu
from jax.experimental.pallas import tpu_sc as plsc

N = 10000
E = 320000
D = 128

NC = 2
NS = 16
NW = NC * NS
K = 128
CT = 80
NP = 5
CTP = CT // NP
EPT = CT * K
NPAD = 10112
RPT = NPAD // NS
PADIDX = 10008

_mesh = plsc.VectorSubcoreMesh(core_axis_name="c", subcore_axis_name="s")


def _make_deg_kernel(degw):
    @functools.partial(
        pl.kernel,
        out_type=jax.ShapeDtypeStruct((NC, NPAD, degw), jnp.float32),
        mesh=_mesh,
        scratch_types=[
            pltpu.VMEM((CT, K), jnp.int32),
            pltpu.VMEM((K, degw), jnp.float32),
            pltpu.VMEM_SHARED((NPAD, degw), jnp.float32),
        ],
    )
    def _deg_kernel(dst_hbm, ones_hbm, zeros_hbm, degp_hbm, idx_v, ones_v,
                    counts):
        cid = lax.axis_index("c")
        sid = lax.axis_index("s")
        w = cid * NS + sid
        pltpu.sync_copy(dst_hbm.at[w], idx_v)
        pltpu.sync_copy(ones_hbm, ones_v)
        pltpu.sync_copy(zeros_hbm, counts.at[pl.ds(sid * RPT, RPT)])
        plsc.subcore_barrier()

        def body(j, carry):
            pltpu.sync_copy(ones_v, counts.at[idx_v.at[j]], add=True)
            return carry

        lax.fori_loop(0, CT, body, 0)
        plsc.subcore_barrier()
        pltpu.sync_copy(counts.at[pl.ds(sid * RPT, RPT)],
                        degp_hbm.at[cid, pl.ds(sid * RPT, RPT)])

    return _deg_kernel


DEGW = 128
_deg_kernel = _make_deg_kernel(DEGW)


@functools.partial(
    pl.kernel,
    out_type=jax.ShapeDtypeStruct((NC, NPAD, D), jnp.float32),
    mesh=_mesh,
    scratch_types=[
        pltpu.VMEM((CTP, K), jnp.int32),
        pltpu.VMEM((CTP, K), jnp.int32),
        pltpu.VMEM((K, D), jnp.float32),
        pltpu.VMEM((K, D), jnp.float32),
        pltpu.VMEM_SHARED((NPAD, D), jnp.float32),
        pltpu.SemaphoreType.DMA,
        pltpu.SemaphoreType.DMA,
        pltpu.SemaphoreType.DMA,
        pltpu.SemaphoreType.DMA,
    ],
)
def _scatter_kernel(y_hbm, src_hbm, dst_hbm, zeros_hbm, sp_hbm,
                    src_v, dst_v, rows0, rows1, accum,
                    gsem0, gsem1, ssem0, ssem1):
    cid = lax.axis_index("c")
    sid = lax.axis_index("s")
    w = cid * NS + sid
    pltpu.sync_copy(zeros_hbm, accum.at[pl.ds(sid * RPT, RPT)])
    plsc.subcore_barrier()

    def gather(j, rows, gsem):
        return pltpu.async_copy(y_hbm.at[src_v.at[j]], rows, gsem)

    def scatter(j, rows, ssem):
        return pltpu.async_copy(rows, accum.at[dst_v.at[j]], ssem, add=True)

    def phase(p, carry):
        pltpu.sync_copy(src_hbm.at[w, pl.ds(p * CTP, CTP)], src_v)
        pltpu.sync_copy(dst_hbm.at[w, pl.ds(p * CTP, CTP)], dst_v)
        gather(0, rows0, gsem0)
        gather(1, rows1, gsem1)

        def body(t, c2):
            j0 = 2 * t
            j1 = j0 + 1
            pltpu.make_async_copy(y_hbm.at[src_v.at[j0]], rows0, gsem0).wait()
            scatter(j0, rows0, ssem0)
            pltpu.make_async_copy(y_hbm.at[src_v.at[j1]], rows1, gsem1).wait()
            scatter(j1, rows1, ssem1)
            pltpu.make_async_copy(rows0, accum.at[dst_v.at[j0]], ssem0).wait()

            @pl.when(j0 + 2 < CTP)
            def _():
                gather(j0 + 2, rows0, gsem0)

            pltpu.make_async_copy(rows1, accum.at[dst_v.at[j1]], ssem1).wait()

            @pl.when(j1 + 2 < CTP)
            def _():
                gather(j1 + 2, rows1, gsem1)

            return c2

        lax.fori_loop(0, CTP // 2, body, 0)
        return carry

    lax.fori_loop(0, NP, phase, 0)
    plsc.subcore_barrier()
    pltpu.sync_copy(accum.at[pl.ds(sid * RPT, RPT)],
                    sp_hbm.at[cid, pl.ds(sid * RPT, RPT)])


def _linear_body(x_ref, w_ref, d0_ref, d1_ref, y_ref, dis_ref):
    deg = d0_ref[:, :1] + d1_ref[:, :1] + 1.0
    dis = lax.rsqrt(deg)
    xw = jnp.dot(x_ref[...], w_ref[...], preferred_element_type=jnp.float32)
    y_ref[...] = xw * dis
    dis_ref[...] = jnp.broadcast_to(dis, dis_ref.shape)


def _final_body(x_ref, y_ref, s0_ref, s1_ref, dis_ref, b_ref, scale_ref, o_ref):
    x = x_ref[...]
    s = s0_ref[...] + s1_ref[...] + y_ref[...]
    conv = dis_ref[:, :1] * s + b_ref[...]
    mn = jnp.sqrt(jnp.sum(conv * conv, axis=-1, keepdims=True))
    msgn = conv / jnp.maximum(mn, 1e-12)
    xn = jnp.sqrt(jnp.sum(x * x, axis=-1, keepdims=True))
    normed = msgn * xn * scale_ref[0, 0]
    o_ref[...] = 0.5 * normed * (1.0 + lax.erf(normed * 0.7071067811865476))


_BLK = 1000
_GRID = N // _BLK


def kernel(X, edge_index, W, b, scale):
    src = edge_index[0]
    dst = edge_index[1]
    pad = ((0, 0), (0, EPT - E // NW))
    src_t = jnp.pad(src.reshape(NW, E // NW), pad, constant_values=PADIDX)
    dst_t = jnp.pad(dst.reshape(NW, E // NW), pad, constant_values=PADIDX)
    src_t = src_t.reshape(NW, CT, K)
    dst_t = dst_t.reshape(NW, CT, K)

    onesw = jnp.ones((K, DEGW), jnp.float32)
    zerosw = jnp.zeros((RPT, DEGW), jnp.float32)
    zerosD = jnp.zeros((RPT, D), jnp.float32)

    degp = _deg_kernel(dst_t, onesw, zerosw)

    d0 = degp[0, :N, :]
    d1 = degp[1, :N, :]
    y, dis = pl.pallas_call(
        _linear_body,
        grid=(_GRID,),
        in_specs=[
            pl.BlockSpec((_BLK, D), lambda i: (i, 0)),
            pl.BlockSpec((D, D), lambda i: (0, 0)),
            pl.BlockSpec((_BLK, DEGW), lambda i: (i, 0)),
            pl.BlockSpec((_BLK, DEGW), lambda i: (i, 0)),
        ],
        out_specs=[
            pl.BlockSpec((_BLK, D), lambda i: (i, 0)),
            pl.BlockSpec((_BLK, 16), lambda i: (i, 0)),
        ],
        out_shape=[
            jax.ShapeDtypeStruct((N, D), jnp.float32),
            jax.ShapeDtypeStruct((N, 16), jnp.float32),
        ],
    )(X, W, d0, d1)

    y_ext = jnp.concatenate([y, jnp.zeros((NPAD - N, D), jnp.float32)], axis=0)

    sp = _scatter_kernel(y_ext, src_t, dst_t, zerosD)

    out = pl.pallas_call(
        _final_body,
        grid=(_GRID,),
        in_specs=[
            pl.BlockSpec((_BLK, D), lambda i: (i, 0)),
            pl.BlockSpec((_BLK, D), lambda i: (i, 0)),
            pl.BlockSpec((_BLK, D), lambda i: (i, 0)),
            pl.BlockSpec((_BLK, D), lambda i: (i, 0)),
            pl.BlockSpec((_BLK, 16), lambda i: (i, 0)),
            pl.BlockSpec((1, D), lambda i: (0, 0)),
            pl.BlockSpec(memory_space=pltpu.SMEM),
        ],
        out_specs=pl.BlockSpec((_BLK, D), lambda i: (i, 0)),
        out_shape=jax.ShapeDtypeStruct((N, D), jnp.float32),
    )(X, y, sp[0, :N, :], sp[1, :N, :], dis,
      b.reshape(1, D), scale.reshape(1, 1))
    return out


# speedup vs baseline: 12.3996x; 1.0544x over previous
import functools

import jax
import jax.numpy as jnp
from jax import lax
from jax.experimental import pallas as pl
from jax.experimental.pallas import tpu as pltpu
from jax.experimental.pallas import tpu_sc as plsc

N = 10000
E = 320000
D = 128

NC = 2
NS = 16
NW = NC * NS
K = 80
CT = 128
NP = 8
CTP = CT // NP
NB = 4
EPT = CT * K
NPAD = 10112
RPT = NPAD // NS
PADIDX = 10008

_mesh = plsc.VectorSubcoreMesh(core_axis_name="c", subcore_axis_name="s")


def _make_deg_kernel(degw):
    @functools.partial(
        pl.kernel,
        out_type=jax.ShapeDtypeStruct((NC, NPAD, degw), jnp.float32),
        mesh=_mesh,
        scratch_types=[
            pltpu.VMEM((CT, K), jnp.int32),
            pltpu.VMEM((K, degw), jnp.float32),
            pltpu.VMEM_SHARED((NPAD, degw), jnp.float32),
        ],
    )
    def _deg_kernel(dst_hbm, ones_hbm, zeros_hbm, degp_hbm, idx_v, ones_v,
                    counts):
        cid = lax.axis_index("c")
        sid = lax.axis_index("s")
        w = cid * NS + sid
        pltpu.sync_copy(dst_hbm.at[w], idx_v)
        pltpu.sync_copy(ones_hbm, ones_v)
        pltpu.sync_copy(zeros_hbm, counts.at[pl.ds(sid * RPT, RPT)])
        plsc.subcore_barrier()

        def body(j, carry):
            pltpu.sync_copy(ones_v, counts.at[idx_v.at[j]], add=True)
            return carry

        lax.fori_loop(0, CT, body, 0)
        plsc.subcore_barrier()
        pltpu.sync_copy(counts.at[pl.ds(sid * RPT, RPT)],
                        degp_hbm.at[cid, pl.ds(sid * RPT, RPT)])

    return _deg_kernel


DEGW = 128
_deg_kernel = _make_deg_kernel(DEGW)


@functools.partial(
    pl.kernel,
    out_type=jax.ShapeDtypeStruct((NC, NPAD, D), jnp.float32),
    mesh=_mesh,
    scratch_types=[
        pltpu.VMEM((CTP, K), jnp.int32),
        pltpu.VMEM((CTP, K), jnp.int32),
        [pltpu.VMEM((K, D), jnp.float32)] * NB,
        pltpu.VMEM_SHARED((NPAD, D), jnp.float32),
        [pltpu.SemaphoreType.DMA] * NB,
        [pltpu.SemaphoreType.DMA] * NB,
    ],
)
def _scatter_kernel(y_hbm, src_hbm, dst_hbm, zeros_hbm, sp_hbm,
                    src_v, dst_v, rows, accum, gsem, ssem):
    cid = lax.axis_index("c")
    sid = lax.axis_index("s")
    w = cid * NS + sid
    pltpu.sync_copy(zeros_hbm, accum.at[pl.ds(sid * RPT, RPT)])
    plsc.subcore_barrier()

    def gather(j, b):
        return pltpu.async_copy(y_hbm.at[src_v.at[j]], rows[b], gsem[b])

    def phase(p, carry):
        pltpu.sync_copy(src_hbm.at[w, pl.ds(p * CTP, CTP)], src_v)
        pltpu.sync_copy(dst_hbm.at[w, pl.ds(p * CTP, CTP)], dst_v)
        for b in range(NB):
            gather(b, b)

        def body(t, c2):
            for b in range(NB):
                j = NB * t + b
                pltpu.make_async_copy(y_hbm.at[src_v.at[j]], rows[b],
                                      gsem[b]).wait()
                pltpu.async_copy(rows[b], accum.at[dst_v.at[j]], ssem[b],
                                 add=True)
                pltpu.make_async_copy(rows[b], accum.at[dst_v.at[j]],
                                      ssem[b]).wait()

                @pl.when(j + NB < CTP)
                def _():
                    gather(j + NB, b)

            return c2

        lax.fori_loop(0, CTP // NB, body, 0)
        return carry

    lax.fori_loop(0, NP, phase, 0)
    plsc.subcore_barrier()
    pltpu.sync_copy(accum.at[pl.ds(sid * RPT, RPT)],
                    sp_hbm.at[cid, pl.ds(sid * RPT, RPT)])


def _linear_body(x_ref, w_ref, d0_ref, d1_ref, y_ref, dis_ref):
    deg = d0_ref[:, :1] + d1_ref[:, :1] + 1.0
    dis = lax.rsqrt(deg)
    xw = jnp.dot(x_ref[...], w_ref[...], preferred_element_type=jnp.float32)
    y_ref[...] = xw * dis
    dis_ref[...] = jnp.broadcast_to(dis, dis_ref.shape)


def _final_body(x_ref, y_ref, s0_ref, s1_ref, dis_ref, b_ref, scale_ref, o_ref):
    x = x_ref[...]
    s = s0_ref[...] + s1_ref[...] + y_ref[...]
    conv = dis_ref[:, :1] * s + b_ref[...]
    mn = jnp.sqrt(jnp.sum(conv * conv, axis=-1, keepdims=True))
    msgn = conv / jnp.maximum(mn, 1e-12)
    xn = jnp.sqrt(jnp.sum(x * x, axis=-1, keepdims=True))
    normed = msgn * xn * scale_ref[0, 0]
    o_ref[...] = 0.5 * normed * (1.0 + lax.erf(normed * 0.7071067811865476))


_BLK = 1000
_GRID = N // _BLK


def kernel(X, edge_index, W, b, scale):
    src = edge_index[0]
    dst = edge_index[1]
    pad = ((0, 0), (0, EPT - E // NW))
    src_t = jnp.pad(src.reshape(NW, E // NW), pad, constant_values=PADIDX)
    dst_t = jnp.pad(dst.reshape(NW, E // NW), pad, constant_values=PADIDX)
    src_t = src_t.reshape(NW, CT, K)
    dst_t = dst_t.reshape(NW, CT, K)

    onesw = jnp.ones((K, DEGW), jnp.float32)
    zerosw = jnp.zeros((RPT, DEGW), jnp.float32)
    zerosD = jnp.zeros((RPT, D), jnp.float32)

    degp = _deg_kernel(dst_t, onesw, zerosw)

    d0 = degp[0, :N, :]
    d1 = degp[1, :N, :]
    y, dis = pl.pallas_call(
        _linear_body,
        grid=(_GRID,),
        in_specs=[
            pl.BlockSpec((_BLK, D), lambda i: (i, 0)),
            pl.BlockSpec((D, D), lambda i: (0, 0)),
            pl.BlockSpec((_BLK, DEGW), lambda i: (i, 0)),
            pl.BlockSpec((_BLK, DEGW), lambda i: (i, 0)),
        ],
        out_specs=[
            pl.BlockSpec((_BLK, D), lambda i: (i, 0)),
            pl.BlockSpec((_BLK, 16), lambda i: (i, 0)),
        ],
        out_shape=[
            jax.ShapeDtypeStruct((N, D), jnp.float32),
            jax.ShapeDtypeStruct((N, 16), jnp.float32),
        ],
    )(X, W, d0, d1)

    y_ext = jnp.concatenate([y, jnp.zeros((NPAD - N, D), jnp.float32)], axis=0)

    sp = _scatter_kernel(y_ext, src_t, dst_t, zerosD)

    out = pl.pallas_call(
        _final_body,
        grid=(_GRID,),
        in_specs=[
            pl.BlockSpec((_BLK, D), lambda i: (i, 0)),
            pl.BlockSpec((_BLK, D), lambda i: (i, 0)),
            pl.BlockSpec((_BLK, D), lambda i: (i, 0)),
            pl.BlockSpec((_BLK, D), lambda i: (i, 0)),
            pl.BlockSpec((_BLK, 16), lambda i: (i, 0)),
            pl.BlockSpec((1, D), lambda i: (0, 0)),
            pl.BlockSpec(memory_space=pltpu.SMEM),
        ],
        out_specs=pl.BlockSpec((_BLK, D), lambda i: (i, 0)),
        out_shape=jax.ShapeDtypeStruct((N, D), jnp.float32),
    )(X, y, sp[0, :N, :], sp[1, :N, :], dis,
      b.reshape(1, D), scale.reshape(1, 1))
    return out


# speedup vs baseline: 13.9600x; 1.1258x over previous
import functools

import jax
import jax.numpy as jnp
from jax import lax
from jax.experimental import pallas as pl
from jax.experimental.pallas import tpu as pltpu
from jax.experimental.pallas import tpu_sc as plsc

N = 10000
E = 320000
D = 128

NC = 2
NS = 16
NW = NC * NS
K = 80
CT = 128
NP = 8
CTP = CT // NP
NB = 4
EPT = CT * K
NPAD = 10112
RPT = NPAD // NS
PADIDX = 10008

_mesh = plsc.VectorSubcoreMesh(core_axis_name="c", subcore_axis_name="s")


DR = EPT // 128
NV = EPT // 16


@functools.partial(
    pl.kernel,
    out_type=jax.ShapeDtypeStruct((NC, DR, 128), jnp.float32),
    mesh=_mesh,
    compiler_params=pltpu.CompilerParams(needs_layout_passes=False),
    scratch_types=[
        pltpu.VMEM((EPT,), jnp.int32),
        pltpu.VMEM((DR, 128), jnp.float32),
        pltpu.VMEM((DR,), jnp.int32),
        pltpu.VMEM_SHARED((DR, 128), jnp.float32),
    ],
)
def _deg_kernel(dst_hbm, degp_hbm, idx_v, counts_v, rowidx_v, counts_sh):
    cid = lax.axis_index("c")
    sid = lax.axis_index("s")
    w = cid * NS + sid
    pltpu.sync_copy(dst_hbm.at[w], idx_v)

    def zero(i, c):
        counts_v[i >> 3, pl.ds((i & 7) * 16, 16)] = jnp.zeros((16,),
                                                              jnp.float32)
        return c

    lax.fori_loop(0, NV, zero, 0)
    for i in range(DR // 16):
        rowidx_v[pl.ds(16 * i, 16)] = lax.iota(jnp.int32, 16) + 16 * i

    @pl.when(sid == 0)
    def _():
        pltpu.sync_copy(counts_v, counts_sh)

    plsc.subcore_barrier()

    ones16 = jnp.ones((16,), jnp.float32)

    def count(i, c):
        idx16 = idx_v[pl.ds(i * 16, 16)]
        row16 = lax.shift_right_logical(idx16, 7)
        col16 = lax.bitwise_and(idx16, 127)
        plsc.addupdate_scatter(counts_v, [row16, col16], ones16)
        return c

    lax.fori_loop(0, NV, count, 0)
    plsc.subcore_barrier()
    pltpu.sync_copy(counts_v, counts_sh.at[rowidx_v], add=True)
    plsc.subcore_barrier()

    @pl.when(sid == 0)
    def _():
        pltpu.sync_copy(counts_sh, degp_hbm.at[cid])


@functools.partial(
    pl.kernel,
    out_type=jax.ShapeDtypeStruct((NC, NPAD, D), jnp.float32),
    mesh=_mesh,
    scratch_types=[
        pltpu.VMEM((CTP, K), jnp.int32),
        pltpu.VMEM((CTP, K), jnp.int32),
        [pltpu.VMEM((K, D), jnp.float32)] * NB,
        pltpu.VMEM_SHARED((NPAD, D), jnp.float32),
        [pltpu.SemaphoreType.DMA] * NB,
        [pltpu.SemaphoreType.DMA] * NB,
    ],
)
def _scatter_kernel(y_hbm, src_hbm, dst_hbm, zeros_hbm, sp_hbm,
                    src_v, dst_v, rows, accum, gsem, ssem):
    cid = lax.axis_index("c")
    sid = lax.axis_index("s")
    w = cid * NS + sid
    pltpu.sync_copy(zeros_hbm, accum.at[pl.ds(sid * RPT, RPT)])
    plsc.subcore_barrier()

    def gather(j, b):
        return pltpu.async_copy(y_hbm.at[src_v.at[j]], rows[b], gsem[b])

    def phase(p, carry):
        pltpu.sync_copy(src_hbm.at[w, pl.ds(p * CTP, CTP)], src_v)
        pltpu.sync_copy(dst_hbm.at[w, pl.ds(p * CTP, CTP)], dst_v)
        for b in range(NB):
            gather(b, b)

        def body(t, c2):
            for b in range(NB):
                j = NB * t + b
                pltpu.make_async_copy(y_hbm.at[src_v.at[j]], rows[b],
                                      gsem[b]).wait()
                pltpu.async_copy(rows[b], accum.at[dst_v.at[j]], ssem[b],
                                 add=True)
                pltpu.make_async_copy(rows[b], accum.at[dst_v.at[j]],
                                      ssem[b]).wait()

                @pl.when(j + NB < CTP)
                def _():
                    gather(j + NB, b)

            return c2

        lax.fori_loop(0, CTP // NB, body, 0)
        return carry

    lax.fori_loop(0, NP, phase, 0)
    plsc.subcore_barrier()
    pltpu.sync_copy(accum.at[pl.ds(sid * RPT, RPT)],
                    sp_hbm.at[cid, pl.ds(sid * RPT, RPT)])


def _linear_body(x_ref, w_ref, d0_ref, d1_ref, y_ref, dis_ref):
    deg = d0_ref[:, :1] + d1_ref[:, :1] + 1.0
    dis = lax.rsqrt(deg)
    xw = jnp.dot(x_ref[...], w_ref[...], preferred_element_type=jnp.float32)
    y_ref[...] = xw * dis
    dis_ref[...] = jnp.broadcast_to(dis, dis_ref.shape)


def _final_body(x_ref, y_ref, s0_ref, s1_ref, dis_ref, b_ref, scale_ref, o_ref):
    x = x_ref[...]
    s = s0_ref[...] + s1_ref[...] + y_ref[...]
    conv = dis_ref[:, :1] * s + b_ref[...]
    mn = jnp.sqrt(jnp.sum(conv * conv, axis=-1, keepdims=True))
    msgn = conv / jnp.maximum(mn, 1e-12)
    xn = jnp.sqrt(jnp.sum(x * x, axis=-1, keepdims=True))
    normed = msgn * xn * scale_ref[0, 0]
    o_ref[...] = 0.5 * normed * (1.0 + lax.erf(normed * 0.7071067811865476))


_BLK = 1000
_GRID = N // _BLK


def kernel(X, edge_index, W, b, scale):
    src = edge_index[0]
    dst = edge_index[1]
    pad = ((0, 0), (0, EPT - E // NW))
    src_t = jnp.pad(src.reshape(NW, E // NW), pad, constant_values=PADIDX)
    dst_t = jnp.pad(dst.reshape(NW, E // NW), pad, constant_values=PADIDX)
    dst_flat = dst_t
    src_t = src_t.reshape(NW, CT, K)
    dst_t = dst_t.reshape(NW, CT, K)

    zerosD = jnp.zeros((RPT, D), jnp.float32)

    degp = _deg_kernel(dst_flat)

    d0 = degp[0].reshape(EPT)[:N, None]
    d1 = degp[1].reshape(EPT)[:N, None]
    y, dis = pl.pallas_call(
        _linear_body,
        grid=(_GRID,),
        in_specs=[
            pl.BlockSpec((_BLK, D), lambda i: (i, 0)),
            pl.BlockSpec((D, D), lambda i: (0, 0)),
            pl.BlockSpec((_BLK, 1), lambda i: (i, 0)),
            pl.BlockSpec((_BLK, 1), lambda i: (i, 0)),
        ],
        out_specs=[
            pl.BlockSpec((_BLK, D), lambda i: (i, 0)),
            pl.BlockSpec((_BLK, 16), lambda i: (i, 0)),
        ],
        out_shape=[
            jax.ShapeDtypeStruct((N, D), jnp.float32),
            jax.ShapeDtypeStruct((N, 16), jnp.float32),
        ],
    )(X, W, d0, d1)

    y_ext = jnp.concatenate([y, jnp.zeros((NPAD - N, D), jnp.float32)], axis=0)

    sp = _scatter_kernel(y_ext, src_t, dst_t, zerosD)

    out = pl.pallas_call(
        _final_body,
        grid=(_GRID,),
        in_specs=[
            pl.BlockSpec((_BLK, D), lambda i: (i, 0)),
            pl.BlockSpec((_BLK, D), lambda i: (i, 0)),
            pl.BlockSpec((_BLK, D), lambda i: (i, 0)),
            pl.BlockSpec((_BLK, D), lambda i: (i, 0)),
            pl.BlockSpec((_BLK, 16), lambda i: (i, 0)),
            pl.BlockSpec((1, D), lambda i: (0, 0)),
            pl.BlockSpec(memory_space=pltpu.SMEM),
        ],
        out_specs=pl.BlockSpec((_BLK, D), lambda i: (i, 0)),
        out_shape=jax.ShapeDtypeStruct((N, D), jnp.float32),
    )(X, y, sp[0, :N, :], sp[1, :N, :], dis,
      b.reshape(1, D), scale.reshape(1, 1))
    return out


# speedup vs baseline: 14.3249x; 1.0261x over previous
import functools

import jax
import jax.numpy as jnp
from jax import lax
from jax.experimental import pallas as pl
from jax.experimental.pallas import tpu as pltpu
from jax.experimental.pallas import tpu_sc as plsc

N = 10000
E = 320000
D = 128

NC = 2
NS = 16
NW = NC * NS
K = 80
CT = 128
NP = 8
CTP = CT // NP
NB = 4
EPT = CT * K
NPAD = 10112
RPT = NPAD // NS
PADIDX = 10008

_mesh = plsc.VectorSubcoreMesh(core_axis_name="c", subcore_axis_name="s")


DR = EPT // 128
NV = EPT // 16


@functools.partial(
    pl.kernel,
    out_type=jax.ShapeDtypeStruct((NC, DR, 128), jnp.float32),
    mesh=_mesh,
    compiler_params=pltpu.CompilerParams(needs_layout_passes=False),
    scratch_types=[
        pltpu.VMEM((EPT,), jnp.int32),
        pltpu.VMEM((DR, 128), jnp.float32),
        pltpu.VMEM((DR,), jnp.int32),
        pltpu.VMEM_SHARED((DR, 128), jnp.float32),
    ],
)
def _deg_kernel(dst_hbm, degp_hbm, idx_v, counts_v, rowidx_v, counts_sh):
    cid = lax.axis_index("c")
    sid = lax.axis_index("s")
    w = cid * NS + sid
    pltpu.sync_copy(dst_hbm.at[w], idx_v)

    def zero(i, c):
        counts_v[i >> 3, pl.ds((i & 7) * 16, 16)] = jnp.zeros((16,),
                                                              jnp.float32)
        return c

    lax.fori_loop(0, NV, zero, 0)
    for i in range(DR // 16):
        rowidx_v[pl.ds(16 * i, 16)] = lax.iota(jnp.int32, 16) + 16 * i

    @pl.when(sid == 0)
    def _():
        pltpu.sync_copy(counts_v, counts_sh)

    plsc.subcore_barrier()

    ones16 = jnp.ones((16,), jnp.float32)

    def count(i, c):
        idx16 = idx_v[pl.ds(i * 16, 16)]
        row16 = lax.shift_right_logical(idx16, 7)
        col16 = lax.bitwise_and(idx16, 127)
        plsc.addupdate_scatter(counts_v, [row16, col16], ones16)
        return c

    lax.fori_loop(0, NV, count, 0)
    plsc.subcore_barrier()
    pltpu.sync_copy(counts_v, counts_sh.at[rowidx_v], add=True)
    plsc.subcore_barrier()

    @pl.when(sid == 0)
    def _():
        pltpu.sync_copy(counts_sh, degp_hbm.at[cid])


@functools.partial(
    pl.kernel,
    out_type=jax.ShapeDtypeStruct((NC, NPAD, D), jnp.float32),
    mesh=_mesh,
    scratch_types=[
        pltpu.VMEM((CTP, K), jnp.int32),
        pltpu.VMEM((CTP, K), jnp.int32),
        [pltpu.VMEM((K, D), jnp.float32)] * NB,
        pltpu.VMEM_SHARED((NPAD, D), jnp.float32),
        [pltpu.SemaphoreType.DMA] * NB,
        [pltpu.SemaphoreType.DMA] * NB,
    ],
)
def _scatter_kernel(y_hbm, src_hbm, dst_hbm, zeros_hbm, sp_hbm,
                    src_v, dst_v, rows, accum, gsem, ssem):
    cid = lax.axis_index("c")
    sid = lax.axis_index("s")
    w = cid * NS + sid
    pltpu.sync_copy(zeros_hbm, accum.at[pl.ds(sid * RPT, RPT)])
    plsc.subcore_barrier()

    def gather(j, b):
        return pltpu.async_copy(y_hbm.at[src_v.at[j]], rows[b], gsem[b])

    def phase(p, carry):
        pltpu.sync_copy(src_hbm.at[w, pl.ds(p * CTP, CTP)], src_v)
        pltpu.sync_copy(dst_hbm.at[w, pl.ds(p * CTP, CTP)], dst_v)
        for b in range(NB):
            gather(b, b)

        def body(t, c2):
            for b in range(NB):
                j = NB * t + b
                pltpu.make_async_copy(y_hbm.at[src_v.at[j]], rows[b],
                                      gsem[b]).wait()
                pltpu.async_copy(rows[b], accum.at[dst_v.at[j]], ssem[b],
                                 add=True)
                pltpu.make_async_copy(rows[b], accum.at[dst_v.at[j]],
                                      ssem[b]).wait()

                @pl.when(j + NB < CTP)
                def _():
                    gather(j + NB, b)

            return c2

        lax.fori_loop(0, CTP // NB, body, 0)
        return carry

    lax.fori_loop(0, NP, phase, 0)
    plsc.subcore_barrier()
    pltpu.sync_copy(accum.at[pl.ds(sid * RPT, RPT)],
                    sp_hbm.at[cid, pl.ds(sid * RPT, RPT)])


def _linear_body(x_ref, w_ref, d0_ref, d1_ref, y_ref, dis_ref):
    deg = d0_ref[:, :1] + d1_ref[:, :1] + 1.0
    dis = lax.rsqrt(deg)
    xw = jnp.dot(x_ref[...], w_ref[...], preferred_element_type=jnp.float32)
    y_ref[...] = xw * dis
    dis_ref[...] = jnp.broadcast_to(dis, dis_ref.shape)


def _final_body(x_ref, y_ref, s0_ref, s1_ref, dis_ref, b_ref, scale_ref, o_ref):
    x = x_ref[...]
    s = s0_ref[0] + s1_ref[0] + y_ref[...]
    conv = dis_ref[:, :1] * s + b_ref[...]
    mn = jnp.sqrt(jnp.sum(conv * conv, axis=-1, keepdims=True))
    msgn = conv / jnp.maximum(mn, 1e-12)
    xn = jnp.sqrt(jnp.sum(x * x, axis=-1, keepdims=True))
    normed = msgn * xn * scale_ref[0, 0]
    o_ref[...] = 0.5 * normed * (1.0 + lax.erf(normed * 0.7071067811865476))


_BLK = 1000
_GRID = N // _BLK


def kernel(X, edge_index, W, b, scale):
    src = edge_index[0]
    dst = edge_index[1]
    pad = ((0, 0), (0, EPT - E // NW))
    src_t = jnp.pad(src.reshape(NW, E // NW), pad, constant_values=PADIDX)
    dst_t = jnp.pad(dst.reshape(NW, E // NW), pad, constant_values=PADIDX)
    dst_flat = dst_t
    src_t = src_t.reshape(NW, CT, K)
    dst_t = dst_t.reshape(NW, CT, K)

    zerosD = jnp.zeros((RPT, D), jnp.float32)

    degp = _deg_kernel(dst_flat)

    d0 = degp[0].reshape(EPT)[:N, None]
    d1 = degp[1].reshape(EPT)[:N, None]
    y, dis = pl.pallas_call(
        _linear_body,
        grid=(_GRID,),
        in_specs=[
            pl.BlockSpec((_BLK, D), lambda i: (i, 0)),
            pl.BlockSpec((D, D), lambda i: (0, 0)),
            pl.BlockSpec((_BLK, 1), lambda i: (i, 0)),
            pl.BlockSpec((_BLK, 1), lambda i: (i, 0)),
        ],
        out_specs=[
            pl.BlockSpec((_BLK, D), lambda i: (i, 0)),
            pl.BlockSpec((_BLK, 16), lambda i: (i, 0)),
        ],
        out_shape=[
            jax.ShapeDtypeStruct((NPAD, D), jnp.float32),
            jax.ShapeDtypeStruct((N, 16), jnp.float32),
        ],
    )(X, W, d0, d1)

    sp = _scatter_kernel(y, src_t, dst_t, zerosD)

    out = pl.pallas_call(
        _final_body,
        grid=(_GRID,),
        in_specs=[
            pl.BlockSpec((_BLK, D), lambda i: (i, 0)),
            pl.BlockSpec((_BLK, D), lambda i: (i, 0)),
            pl.BlockSpec((1, _BLK, D), lambda i: (0, i, 0)),
            pl.BlockSpec((1, _BLK, D), lambda i: (1, i, 0)),
            pl.BlockSpec((_BLK, 16), lambda i: (i, 0)),
            pl.BlockSpec((1, D), lambda i: (0, 0)),
            pl.BlockSpec(memory_space=pltpu.SMEM),
        ],
        out_specs=pl.BlockSpec((_BLK, D), lambda i: (i, 0)),
        out_shape=jax.ShapeDtypeStruct((N, D), jnp.float32),
    )(X, y, sp, sp, dis, b.reshape(1, D), scale.reshape(1, 1))
    return out
